# baseline (speedup 1.0000x reference)
import functools

import jax
import jax.numpy as jnp
from jax import lax
from jax.experimental import pallas as pl
from jax.experimental.pallas import tpu as pltpu
from jax.experimental.pallas import tpu_sc as plsc

F_IN, R, C = 32, 64, 2048
HID, F_OUT = 64, 32
S, N_IND, M_MIX = 2, 2048, 4096
NSET = S * 3
NU = NSET * M_MIX
ROWW = R * F_IN
CP = C + 8
NW = 32
PER_W = NU // NW
GCHUNK = 32
RW32 = ROWW // 2
DROWS = S * M_MIX
COLS_PER_TILE = 32
NPASS = C // (NW * COLS_PER_TILE)


def _k1_body(a_l_ref, b_l_ref, b_c_ref, q_c_ref, src_ref, dst_ref, inv_ref):
    a_l = a_l_ref[0].astype(jnp.float32)
    b_l = b_l_ref[0].astype(jnp.float32)
    j_l = lax.broadcasted_iota(jnp.int32, (1, N_IND), 1).astype(jnp.float32)
    PB = 256
    for pb in range(N_IND // PB):
        p_c = lax.broadcasted_iota(jnp.int32, (PB, 1), 0).astype(jnp.float32) + float(pb * PB)
        bmat = jnp.where(a_l == p_c, j_l + 1.0, 0.0)
        jstar = jnp.max(bmat, axis=1, keepdims=True) - 1.0
        val = jnp.sum(jnp.where(j_l == jstar, b_l, 0.0), axis=1, keepdims=True)
        inv_ref[pl.ds(pb * PB, PB), :] = jnp.where(jstar < 0.0, float(C), val)
    inv_c = inv_ref[...]
    b_c = b_c_ref[0].astype(jnp.float32)
    tab = jnp.concatenate([inv_c, b_c], axis=1)
    p_l = lax.broadcasted_iota(jnp.int32, (1, N_IND), 1).astype(jnp.float32)
    MB = 512
    for mb in range(M_MIX // MB):
        q_c = q_c_ref[0, pl.ds(mb * MB, MB), :].astype(jnp.float32)
        onehot = (q_c == p_l).astype(jnp.float32)
        res = jnp.dot(onehot, tab, preferred_element_type=jnp.float32,
                      precision=jax.lax.Precision.HIGHEST)
        src_ref[0, pl.ds(mb * MB, MB), :] = res[:, 0:1].astype(jnp.int32)
        dst_ref[0, pl.ds(mb * MB, MB), :] = res[:, 1:2].astype(jnp.int32)


def _k1_call(a_l, b_l, b_c, q_c):
    return pl.pallas_call(
        _k1_body,
        grid=(NSET,),
        in_specs=[
            pl.BlockSpec((1, 1, N_IND), lambda g: (g, 0, 0)),
            pl.BlockSpec((1, 1, N_IND), lambda g: (g, 0, 0)),
            pl.BlockSpec((1, N_IND, 1), lambda g: (g, 0, 0)),
            pl.BlockSpec((1, M_MIX, 1), lambda g: (g, 0, 0)),
        ],
        out_specs=[
            pl.BlockSpec((1, M_MIX, 1), lambda g: (g, 0, 0)),
            pl.BlockSpec((1, M_MIX, 1), lambda g: (g, 0, 0)),
        ],
        out_shape=[
            jax.ShapeDtypeStruct((NSET, M_MIX, 1), jnp.int32),
            jax.ShapeDtypeStruct((NSET, M_MIX, 1), jnp.int32),
        ],
        scratch_shapes=[pltpu.VMEM((N_IND, 1), jnp.float32)],
    )(a_l, b_l, b_c, q_c)


def _k2_body(tin_hbm, src_hbm, x_hbm, idx_v, buf_v, sem):
    wid = lax.axis_index("s") * 2 + lax.axis_index("c")
    base = wid * PER_W
    pltpu.sync_copy(src_hbm.at[pl.ds(base, PER_W)], idx_v)

    def step(t, _):
        pltpu.async_copy(
            tin_hbm.at[idx_v.at[pl.ds(t * GCHUNK, GCHUNK)]], buf_v, sem
        ).wait()
        pltpu.sync_copy(buf_v, x_hbm.at[pl.ds(base + t * GCHUNK, GCHUNK)])
        return 0

    lax.fori_loop(0, PER_W // GCHUNK, step, 0)


def _k2_call(tin, src):
    mesh = plsc.VectorSubcoreMesh(core_axis_name="c", subcore_axis_name="s")
    f = functools.partial(
        pl.kernel,
        mesh=mesh,
        out_type=jax.ShapeDtypeStruct((NU, RW32), jnp.int32),
        scratch_types=[
            pltpu.VMEM((PER_W,), jnp.int32),
            pltpu.VMEM((GCHUNK, RW32), jnp.int32),
            pltpu.SemaphoreType.DMA,
        ],
        compiler_params=pltpu.CompilerParams(needs_layout_passes=False),
    )(_k2_body)
    return f(tin, src)


def _k3_body(x0_ref, x1_ref, x2_ref, w1t_ref, b1_ref, w2t_ref, b2_ref, d_ref):
    x0 = x0_ref[0, 0]
    x1 = x1_ref[0, 0]
    x2 = x2_ref[0, 0]
    h = jnp.dot(x0, w1t_ref[0:F_IN, :], preferred_element_type=jnp.float32)
    h += jnp.dot(x1, w1t_ref[F_IN:2 * F_IN, :], preferred_element_type=jnp.float32)
    h += jnp.dot(x2, w1t_ref[2 * F_IN:3 * F_IN, :], preferred_element_type=jnp.float32)
    a = jnp.maximum(h + b1_ref[...], 0.0).astype(jnp.bfloat16)
    d = jnp.dot(a, w2t_ref[...], preferred_element_type=jnp.float32) + b2_ref[...]
    d_ref[0] = d.astype(jnp.bfloat16)


def _k3_call(xr, w1t, b1r, w2t, b2r):
    BR = 8192
    NB = (M_MIX * R) // BR
    return pl.pallas_call(
        _k3_body,
        grid=(S, NB),
        in_specs=[
            pl.BlockSpec((1, 1, BR, F_IN), lambda s, t: (s, 0, t, 0)),
            pl.BlockSpec((1, 1, BR, F_IN), lambda s, t: (s, 1, t, 0)),
            pl.BlockSpec((1, 1, BR, F_IN), lambda s, t: (s, 2, t, 0)),
            pl.BlockSpec((3 * F_IN, HID), lambda s, t: (0, 0)),
            pl.BlockSpec((1, HID), lambda s, t: (0, 0)),
            pl.BlockSpec((HID, F_OUT), lambda s, t: (0, 0)),
            pl.BlockSpec((1, F_OUT), lambda s, t: (0, 0)),
        ],
        out_specs=pl.BlockSpec((1, BR, F_OUT), lambda s, t: (s, t, 0)),
        out_shape=jax.ShapeDtypeStruct((S, M_MIX * R, F_OUT), jnp.bfloat16),
    )(xr, xr, xr, w1t, b1r, w2t, b2r)


def _k4_body(d_hbm, dst_hbm, out_hbm, acc_v, dch_v, ul_v, gbuf_v, sem):
    wid = lax.axis_index("s") * 2 + lax.axis_index("c")
    lane = lax.iota(jnp.int32, 16)
    DCH = 2048
    ACC = COLS_PER_TILE * RW32

    def one_pass(p, _):
        c0 = p * (NW * COLS_PER_TILE) + wid * COLS_PER_TILE

        def zero_step(k, _):
            acc_v[pl.ds(k * 16, 16)] = jnp.zeros((16,), jnp.int32)
            return 0

        lax.fori_loop(0, ACC // 16, zero_step, 0)

        def comp_chunk(ch, cnt):
            pltpu.sync_copy(dst_hbm.at[pl.ds(ch * DCH, DCH)], dch_v)

            def comp_vec(j, cnt):
                d16 = dch_v[pl.ds(j * 16, 16)]
                rel = d16 - c0
                msk = (rel >= 0) & (rel < COLS_PER_TILE)
                pc = jnp.cumsum(jnp.where(msk, 1, 0))
                pos = pc + (cnt - 1)
                u = ch * DCH + j * 16 + lane
                plsc.store_scatter(ul_v, [pos], u * 32 + rel, mask=msk)
                return cnt + jnp.max(pc, axis=0)

            return lax.fori_loop(0, DCH // 16, comp_vec, cnt)

        cnt = lax.fori_loop(0, NU // DCH, comp_chunk, jnp.int32(0))

        def rmw_chunk(t, _):
            base = t * 16
            pk = ul_v[pl.ds(base, 16)]
            valid = (base + lane) < cnt
            pk = jnp.where(valid, pk, 0)
            u = pk >> 5
            rel = pk & 31
            row = (u // (3 * M_MIX)) * M_MIX + (u % M_MIX)
            row = jnp.where(valid, row, 0)
            pltpu.async_copy(d_hbm.at[row], gbuf_v, sem).wait()
            for g in range(16):
                @pl.when(base + g < cnt)
                def _():
                    c_g = jnp.max(jnp.where(lane == g, rel, 0), axis=0)
                    off = c_g * RW32

                    def rmw_vec(jj, _):
                        cur = plsc.bitcast(acc_v[pl.ds(off + jj * 16, 16)],
                                           jnp.bfloat16)
                        dv = plsc.bitcast(gbuf_v[g, pl.ds(jj * 16, 16)],
                                          jnp.bfloat16)
                        acc_v[pl.ds(off + jj * 16, 16)] = plsc.bitcast(
                            jnp.maximum(cur, dv), jnp.int32)
                        return 0

                    lax.fori_loop(0, RW32 // 16, rmw_vec, 0)
            return 0

        nch = (cnt + 15) >> 4
        lax.fori_loop(0, nch, rmw_chunk, 0)
        pltpu.sync_copy(acc_v, out_hbm.at[pl.ds(c0 * RW32, ACC)])
        return 0

    lax.fori_loop(0, NPASS, one_pass, 0)


def _k4_call(d, dst):
    mesh = plsc.VectorSubcoreMesh(core_axis_name="c", subcore_axis_name="s")
    f = functools.partial(
        pl.kernel,
        mesh=mesh,
        out_type=jax.ShapeDtypeStruct((C * RW32,), jnp.int32),
        scratch_types=[
            pltpu.VMEM((COLS_PER_TILE * RW32,), jnp.int32),
            pltpu.VMEM((2048,), jnp.int32),
            pltpu.VMEM((NU,), jnp.int32),
            pltpu.VMEM((16, RW32), jnp.int32),
            pltpu.SemaphoreType.DMA,
        ],
        compiler_params=pltpu.CompilerParams(needs_layout_passes=False),
    )(_k4_body)
    return f(d, dst)


def kernel(input_tensor, ind0, ind1, ind2, mix_ind, w1, b1, w2, b2):
    tin = jnp.transpose(input_tensor, (2, 1, 0)).reshape(C, ROWW).astype(jnp.bfloat16)
    tin = jnp.concatenate([tin, jnp.zeros((CP - C, ROWW), jnp.bfloat16)], axis=0)
    tin = lax.bitcast_convert_type(tin.reshape(CP, RW32, 2), jnp.int32)

    inds = jnp.stack([ind0, ind1, ind2], axis=1)
    a_l = inds[:, :, :, 0].reshape(NSET, 1, N_IND)
    b_l = inds[:, :, :, 1].reshape(NSET, 1, N_IND)
    b_c = inds[:, :, :, 1].reshape(NSET, N_IND, 1)
    q_c = mix_ind.reshape(NSET, M_MIX, 1)

    src6, dst6 = _k1_call(a_l, b_l, b_c, q_c)
    src = src6.reshape(NU)
    dst = dst6.reshape(NU)

    x = _k2_call(tin, src)
    xb = lax.bitcast_convert_type(x, jnp.bfloat16)
    xr = xb.reshape(S, 3, M_MIX * R, F_IN)

    w1t = jnp.transpose(w1, (1, 0)).astype(jnp.bfloat16)
    w2t = jnp.transpose(w2, (1, 0)).astype(jnp.bfloat16)
    d = _k3_call(xr, w1t, b1.reshape(1, HID), w2t, b2.reshape(1, F_OUT))
    d2 = lax.bitcast_convert_type(d.reshape(DROWS, RW32, 2), jnp.int32)

    out_flat = _k4_call(d2, dst)
    out_bf = lax.bitcast_convert_type(out_flat, jnp.bfloat16)
    return jnp.transpose(out_bf.reshape(C, R, F_OUT), (2, 1, 0)).astype(jnp.float32)


# speedup vs baseline: 10.7443x; 10.7443x over previous
import functools

import jax
import jax.numpy as jnp
from jax import lax
from jax.experimental import pallas as pl
from jax.experimental.pallas import tpu as pltpu
from jax.experimental.pallas import tpu_sc as plsc

F_IN, R, C = 32, 64, 2048
HID, F_OUT = 64, 32
S, N_IND, M_MIX = 2, 2048, 4096
NSET = S * 3
NU = NSET * M_MIX
ROWW = R * F_IN
CP = C + 8
NW = 32
PER_W = NU // NW
GCHUNK = 24
NCH2 = PER_W // GCHUNK
DROWS = S * M_MIX
COLS_PER_TILE = 16
NPASS = C // (NW * COLS_PER_TILE)


def _k1_body(a_l_ref, b_l_ref, b_c_ref, q_c_ref, src_ref, dst_ref, inv_ref):
    a_l = a_l_ref[0].astype(jnp.float32)
    b_l = b_l_ref[0].astype(jnp.float32)
    j_l = lax.broadcasted_iota(jnp.int32, (1, N_IND), 1).astype(jnp.float32)
    PB = 256
    for pb in range(N_IND // PB):
        p_c = lax.broadcasted_iota(jnp.int32, (PB, 1), 0).astype(jnp.float32) + float(pb * PB)
        bmat = jnp.where(a_l == p_c, j_l + 1.0, 0.0)
        jstar = jnp.max(bmat, axis=1, keepdims=True) - 1.0
        val = jnp.sum(jnp.where(j_l == jstar, b_l, 0.0), axis=1, keepdims=True)
        inv_ref[pl.ds(pb * PB, PB), :] = jnp.where(jstar < 0.0, float(C), val)
    inv_c = inv_ref[...]
    b_c = b_c_ref[0].astype(jnp.float32)
    tab = jnp.concatenate([inv_c, b_c], axis=1)
    p_l = lax.broadcasted_iota(jnp.int32, (1, N_IND), 1).astype(jnp.float32)
    MB = 512
    for mb in range(M_MIX // MB):
        q_c = q_c_ref[0, pl.ds(mb * MB, MB), :].astype(jnp.float32)
        onehot = (q_c == p_l).astype(jnp.float32)
        res = jnp.dot(onehot, tab, preferred_element_type=jnp.float32,
                      precision=jax.lax.Precision.HIGHEST)
        src_ref[0, pl.ds(mb * MB, MB), :] = res[:, 0:1].astype(jnp.int32)
        dst_ref[0, pl.ds(mb * MB, MB), :] = res[:, 1:2].astype(jnp.int32)


def _k1_call(a_l, b_l, b_c, q_c):
    return pl.pallas_call(
        _k1_body,
        grid=(NSET,),
        in_specs=[
            pl.BlockSpec((1, 1, N_IND), lambda g: (g, 0, 0)),
            pl.BlockSpec((1, 1, N_IND), lambda g: (g, 0, 0)),
            pl.BlockSpec((1, N_IND, 1), lambda g: (g, 0, 0)),
            pl.BlockSpec((1, M_MIX, 1), lambda g: (g, 0, 0)),
        ],
        out_specs=[
            pl.BlockSpec((1, M_MIX, 1), lambda g: (g, 0, 0)),
            pl.BlockSpec((1, M_MIX, 1), lambda g: (g, 0, 0)),
        ],
        out_shape=[
            jax.ShapeDtypeStruct((NSET, M_MIX, 1), jnp.int32),
            jax.ShapeDtypeStruct((NSET, M_MIX, 1), jnp.int32),
        ],
        scratch_shapes=[pltpu.VMEM((N_IND, 1), jnp.float32)],
    )(a_l, b_l, b_c, q_c)


def _k2_body(tin_hbm, src_hbm, x_hbm, idx_v, buf0, buf1, gs0, gs1, ws0, ws1):
    wid = lax.axis_index("s") * 2 + lax.axis_index("c")
    base = wid * PER_W
    pltpu.sync_copy(src_hbm.at[pl.ds(base, PER_W)], idx_v)

    def g_start(t, buf, sem):
        pltpu.async_copy(tin_hbm.at[idx_v.at[pl.ds(t * GCHUNK, GCHUNK)]], buf, sem)

    def g_wait(buf, sem):
        pltpu.make_async_copy(x_hbm.at[pl.ds(0, GCHUNK)], buf, sem).wait()

    def w_start(t, buf, sem):
        pltpu.async_copy(buf, x_hbm.at[pl.ds(base + t * GCHUNK, GCHUNK)], sem)

    def w_wait(buf, sem):
        pltpu.make_async_copy(buf, x_hbm.at[pl.ds(0, GCHUNK)], sem).wait()

    g_start(0, buf0, gs0)
    NPAIR = NCH2 // 2

    def pair(q, _):
        t0 = 2 * q

        @pl.when(q > 0)
        def _():
            w_wait(buf1, ws1)

        g_start(t0 + 1, buf1, gs1)
        g_wait(buf0, gs0)
        w_start(t0, buf0, ws0)

        @pl.when(q + 1 < NPAIR)
        def _():
            w_wait(buf0, ws0)
            g_start(t0 + 2, buf0, gs0)

        g_wait(buf1, gs1)
        w_start(t0 + 1, buf1, ws1)
        return 0

    lax.fori_loop(0, NPAIR, pair, 0)
    w_wait(buf0, ws0)
    w_wait(buf1, ws1)


def _k2_call(tin, src):
    mesh = plsc.VectorSubcoreMesh(core_axis_name="c", subcore_axis_name="s")
    f = functools.partial(
        pl.kernel,
        mesh=mesh,
        out_type=jax.ShapeDtypeStruct((NU, ROWW), jnp.float32),
        scratch_types=[
            pltpu.VMEM((PER_W,), jnp.int32),
            pltpu.VMEM((GCHUNK, ROWW), jnp.float32),
            pltpu.VMEM((GCHUNK, ROWW), jnp.float32),
            pltpu.SemaphoreType.DMA,
            pltpu.SemaphoreType.DMA,
            pltpu.SemaphoreType.DMA,
            pltpu.SemaphoreType.DMA,
        ],
        compiler_params=pltpu.CompilerParams(needs_layout_passes=False),
    )(_k2_body)
    return f(tin, src)


def _k3_body(x0_ref, x1_ref, x2_ref, w1t_ref, b1_ref, w2t_ref, b2_ref, d_ref):
    x0 = x0_ref[0, 0].astype(jnp.bfloat16)
    x1 = x1_ref[0, 0].astype(jnp.bfloat16)
    x2 = x2_ref[0, 0].astype(jnp.bfloat16)
    h = jnp.dot(x0, w1t_ref[0:F_IN, :], preferred_element_type=jnp.float32)
    h += jnp.dot(x1, w1t_ref[F_IN:2 * F_IN, :], preferred_element_type=jnp.float32)
    h += jnp.dot(x2, w1t_ref[2 * F_IN:3 * F_IN, :], preferred_element_type=jnp.float32)
    a = jnp.maximum(h + b1_ref[...], 0.0).astype(jnp.bfloat16)
    d = jnp.dot(a, w2t_ref[...], preferred_element_type=jnp.float32) + b2_ref[...]
    d_ref[0] = d


def _k3_call(xr, w1t, b1r, w2t, b2r):
    BR = 8192
    NB = (M_MIX * R) // BR
    return pl.pallas_call(
        _k3_body,
        grid=(S, NB),
        in_specs=[
            pl.BlockSpec((1, 1, BR, F_IN), lambda s, t: (s, 0, t, 0)),
            pl.BlockSpec((1, 1, BR, F_IN), lambda s, t: (s, 1, t, 0)),
            pl.BlockSpec((1, 1, BR, F_IN), lambda s, t: (s, 2, t, 0)),
            pl.BlockSpec((3 * F_IN, HID), lambda s, t: (0, 0)),
            pl.BlockSpec((1, HID), lambda s, t: (0, 0)),
            pl.BlockSpec((HID, F_OUT), lambda s, t: (0, 0)),
            pl.BlockSpec((1, F_OUT), lambda s, t: (0, 0)),
        ],
        out_specs=pl.BlockSpec((1, BR, F_OUT), lambda s, t: (s, t, 0)),
        out_shape=jax.ShapeDtypeStruct((S, M_MIX * R, F_OUT), jnp.float32),
    )(xr, xr, xr, w1t, b1r, w2t, b2r)


def _k4_body(d_hbm, dst_hbm, out_hbm, acc_v, dch_v, ul_v, gb0, gb1, sm0, sm1):
    wid = lax.axis_index("s") * 2 + lax.axis_index("c")
    lane = lax.iota(jnp.int32, 16)
    DCH = 2048
    ACC = COLS_PER_TILE * ROWW

    def one_pass(p, _):
        c0 = p * (NW * COLS_PER_TILE) + wid * COLS_PER_TILE

        @plsc.parallel_loop(0, ACC // 16, 1, unroll=8)
        def _(k):
            acc_v[pl.ds(k * 16, 16)] = jnp.zeros((16,), jnp.float32)

        def comp_chunk(ch, cnt):
            pltpu.sync_copy(dst_hbm.at[pl.ds(ch * DCH, DCH)], dch_v)

            def comp_vec(j, cnt):
                d16 = dch_v[pl.ds(j * 16, 16)]
                rel = d16 - c0
                msk = (rel >= 0) & (rel < COLS_PER_TILE)
                pc = jnp.cumsum(jnp.where(msk, 1, 0))
                pos = pc + (cnt - 1)
                u = ch * DCH + j * 16 + lane
                plsc.store_scatter(ul_v, [pos], u * 32 + rel, mask=msk)
                return cnt + jnp.max(pc, axis=0)

            return lax.fori_loop(0, DCH // 16, comp_vec, cnt)

        cnt = lax.fori_loop(0, NU // DCH, comp_chunk, jnp.int32(0))
        nch = (cnt + 15) >> 4

        def load_rows(t):
            base = t * 16
            pk = ul_v[pl.ds(base, 16)]
            valid = (base + lane) < cnt
            pk = jnp.where(valid, pk, 0)
            u = pk >> 5
            row = (u // (3 * M_MIX)) * M_MIX + (u % M_MIX)
            return jnp.where(valid, row, 0)

        def g_start(t, buf, sem):
            pltpu.async_copy(d_hbm.at[load_rows(t)], buf, sem)

        def g_wait(buf, sem):
            pltpu.make_async_copy(d_hbm.at[pl.ds(0, 16)], buf, sem).wait()

        def rmw_chunk(t, buf):
            base = t * 16
            pk = ul_v[pl.ds(base, 16)]
            rel = pk & 31
            for g in range(16):
                @pl.when(base + g < cnt)
                def _():
                    c_g = jnp.max(jnp.where(lane == g, rel, 0), axis=0)
                    off = c_g * ROWW

                    @plsc.parallel_loop(0, ROWW // 16, 1, unroll=8)
                    def _(jj):
                        cur = acc_v[pl.ds(off + jj * 16, 16)]
                        dv = buf[g, pl.ds(jj * 16, 16)]
                        acc_v[pl.ds(off + jj * 16, 16)] = jnp.maximum(cur, dv)

        @pl.when(nch > 0)
        def _():
            g_start(0, gb0, sm0)

        npair = (nch + 1) >> 1

        def pair(q, _):
            t0 = 2 * q

            @pl.when(t0 + 1 < nch)
            def _():
                g_start(t0 + 1, gb1, sm1)

            g_wait(gb0, sm0)
            rmw_chunk(t0, gb0)

            @pl.when(t0 + 2 < nch)
            def _():
                g_start(t0 + 2, gb0, sm0)

            @pl.when(t0 + 1 < nch)
            def _():
                g_wait(gb1, sm1)
                rmw_chunk(t0 + 1, gb1)

            return 0

        lax.fori_loop(0, npair, pair, 0)
        pltpu.sync_copy(acc_v, out_hbm.at[pl.ds(c0 * ROWW, ACC)])
        return 0

    lax.fori_loop(0, NPASS, one_pass, 0)


def _k4_call(d, dst):
    mesh = plsc.VectorSubcoreMesh(core_axis_name="c", subcore_axis_name="s")
    f = functools.partial(
        pl.kernel,
        mesh=mesh,
        out_type=jax.ShapeDtypeStruct((C * ROWW,), jnp.float32),
        scratch_types=[
            pltpu.VMEM((COLS_PER_TILE * ROWW,), jnp.float32),
            pltpu.VMEM((2048,), jnp.int32),
            pltpu.VMEM((NU,), jnp.int32),
            pltpu.VMEM((16, ROWW), jnp.float32),
            pltpu.VMEM((16, ROWW), jnp.float32),
            pltpu.SemaphoreType.DMA,
            pltpu.SemaphoreType.DMA,
        ],
        compiler_params=pltpu.CompilerParams(needs_layout_passes=False),
    )(_k4_body)
    return f(d, dst)


def kernel(input_tensor, ind0, ind1, ind2, mix_ind, w1, b1, w2, b2):
    tin = jnp.transpose(input_tensor, (2, 1, 0)).reshape(C, ROWW)
    tin = jnp.concatenate([tin, jnp.zeros((CP - C, ROWW), jnp.float32)], axis=0)

    inds = jnp.stack([ind0, ind1, ind2], axis=1)
    a_l = inds[:, :, :, 0].reshape(NSET, 1, N_IND)
    b_l = inds[:, :, :, 1].reshape(NSET, 1, N_IND)
    b_c = inds[:, :, :, 1].reshape(NSET, N_IND, 1)
    q_c = mix_ind.reshape(NSET, M_MIX, 1)

    src6, dst6 = _k1_call(a_l, b_l, b_c, q_c)
    src = src6.reshape(NU)
    dst = dst6.reshape(NU)

    x = _k2_call(tin, src)
    xr = x.reshape(S, 3, M_MIX * R, F_IN)

    w1t = jnp.transpose(w1, (1, 0)).astype(jnp.bfloat16)
    w2t = jnp.transpose(w2, (1, 0)).astype(jnp.bfloat16)
    d = _k3_call(xr, w1t, b1.reshape(1, HID), w2t, b2.reshape(1, F_OUT))
    d2 = d.reshape(DROWS, ROWW)

    out_flat = _k4_call(d2, dst)
    return jnp.transpose(out_flat.reshape(C, R, F_OUT), (2, 1, 0))


# speedup vs baseline: 10.9321x; 1.0175x over previous
import functools

import jax
import jax.numpy as jnp
from jax import lax
from jax.experimental import pallas as pl
from jax.experimental.pallas import tpu as pltpu
from jax.experimental.pallas import tpu_sc as plsc

F_IN, R, C = 32, 64, 2048
HID, F_OUT = 64, 32
S, N_IND, M_MIX = 2, 2048, 4096
NSET = S * 3
NU = NSET * M_MIX
ROWW = R * F_IN
CP = C + 8
NW = 32
PER_W = NU // NW
GCHUNK = 24
NCH2 = PER_W // GCHUNK
DROWS = S * M_MIX
COLS_PER_TILE = 16
NPASS = C // (NW * COLS_PER_TILE)


def _k1_body(a_l_ref, b_l_ref, b_c_ref, q_c_ref, src_ref, dst_ref, inv_ref):
    a_l = a_l_ref[0].astype(jnp.float32)
    b_l = b_l_ref[0].astype(jnp.float32)
    j_l = lax.broadcasted_iota(jnp.int32, (1, N_IND), 1).astype(jnp.float32)
    PB = 256
    for pb in range(N_IND // PB):
        p_c = lax.broadcasted_iota(jnp.int32, (PB, 1), 0).astype(jnp.float32) + float(pb * PB)
        bmat = jnp.where(a_l == p_c, j_l + 1.0, 0.0)
        jstar = jnp.max(bmat, axis=1, keepdims=True) - 1.0
        val = jnp.sum(jnp.where(j_l == jstar, b_l, 0.0), axis=1, keepdims=True)
        inv_ref[pl.ds(pb * PB, PB), :] = jnp.where(jstar < 0.0, float(C), val)
    inv_c = inv_ref[...]
    b_c = b_c_ref[0].astype(jnp.float32)
    tab = jnp.concatenate([inv_c, b_c], axis=1)
    p_l = lax.broadcasted_iota(jnp.int32, (1, N_IND), 1).astype(jnp.float32)
    MB = 512
    for mb in range(M_MIX // MB):
        q_c = q_c_ref[0, pl.ds(mb * MB, MB), :].astype(jnp.float32)
        onehot = (q_c == p_l).astype(jnp.float32)
        res = jnp.dot(onehot, tab, preferred_element_type=jnp.float32,
                      precision=jax.lax.Precision.HIGHEST)
        src_ref[0, pl.ds(mb * MB, MB), :] = res[:, 0:1].astype(jnp.int32)
        dst_ref[0, pl.ds(mb * MB, MB), :] = res[:, 1:2].astype(jnp.int32)


def _k1_call(a_l, b_l, b_c, q_c):
    return pl.pallas_call(
        _k1_body,
        grid=(NSET,),
        in_specs=[
            pl.BlockSpec((1, 1, N_IND), lambda g: (g, 0, 0)),
            pl.BlockSpec((1, 1, N_IND), lambda g: (g, 0, 0)),
            pl.BlockSpec((1, N_IND, 1), lambda g: (g, 0, 0)),
            pl.BlockSpec((1, M_MIX, 1), lambda g: (g, 0, 0)),
        ],
        out_specs=[
            pl.BlockSpec((1, M_MIX, 1), lambda g: (g, 0, 0)),
            pl.BlockSpec((1, M_MIX, 1), lambda g: (g, 0, 0)),
        ],
        out_shape=[
            jax.ShapeDtypeStruct((NSET, M_MIX, 1), jnp.int32),
            jax.ShapeDtypeStruct((NSET, M_MIX, 1), jnp.int32),
        ],
        scratch_shapes=[pltpu.VMEM((N_IND, 1), jnp.float32)],
    )(a_l, b_l, b_c, q_c)


def _k2_body(tin_hbm, src_hbm, x_hbm, idx_v, buf0, buf1, gs0, gs1, ws0, ws1):
    wid = lax.axis_index("s") * 2 + lax.axis_index("c")
    base = wid * PER_W
    pltpu.sync_copy(src_hbm.at[pl.ds(base, PER_W)], idx_v)

    def g_start(t, buf, sem):
        pltpu.async_copy(tin_hbm.at[idx_v.at[pl.ds(t * GCHUNK, GCHUNK)]], buf, sem)

    def g_wait(buf, sem):
        pltpu.make_async_copy(x_hbm.at[pl.ds(0, GCHUNK)], buf, sem).wait()

    def w_start(t, buf, sem):
        pltpu.async_copy(buf, x_hbm.at[pl.ds(base + t * GCHUNK, GCHUNK)], sem)

    def w_wait(buf, sem):
        pltpu.make_async_copy(buf, x_hbm.at[pl.ds(0, GCHUNK)], sem).wait()

    g_start(0, buf0, gs0)
    NPAIR = NCH2 // 2

    def pair(q, _):
        t0 = 2 * q

        @pl.when(q > 0)
        def _():
            w_wait(buf1, ws1)

        g_start(t0 + 1, buf1, gs1)
        g_wait(buf0, gs0)
        w_start(t0, buf0, ws0)

        @pl.when(q + 1 < NPAIR)
        def _():
            w_wait(buf0, ws0)
            g_start(t0 + 2, buf0, gs0)

        g_wait(buf1, gs1)
        w_start(t0 + 1, buf1, ws1)
        return 0

    lax.fori_loop(0, NPAIR, pair, 0)
    w_wait(buf0, ws0)
    w_wait(buf1, ws1)


def _k2_call(tin, src):
    mesh = plsc.VectorSubcoreMesh(core_axis_name="c", subcore_axis_name="s")
    f = functools.partial(
        pl.kernel,
        mesh=mesh,
        out_type=jax.ShapeDtypeStruct((NU, 16, 128), jnp.float32),
        scratch_types=[
            pltpu.VMEM((PER_W,), jnp.int32),
            pltpu.VMEM((GCHUNK, 16, 128), jnp.float32),
            pltpu.VMEM((GCHUNK, 16, 128), jnp.float32),
            pltpu.SemaphoreType.DMA,
            pltpu.SemaphoreType.DMA,
            pltpu.SemaphoreType.DMA,
            pltpu.SemaphoreType.DMA,
        ],
        compiler_params=pltpu.CompilerParams(needs_layout_passes=False),
    )(_k2_body)
    return f(tin, src)


def _k3_body(x0_ref, x1_ref, x2_ref, w1t_ref, b1_ref, w2t_ref, b2_ref, d_ref):
    x0 = x0_ref[0, 0].astype(jnp.bfloat16)
    x1 = x1_ref[0, 0].astype(jnp.bfloat16)
    x2 = x2_ref[0, 0].astype(jnp.bfloat16)
    h = jnp.dot(x0, w1t_ref[0:F_IN, :], preferred_element_type=jnp.float32)
    h += jnp.dot(x1, w1t_ref[F_IN:2 * F_IN, :], preferred_element_type=jnp.float32)
    h += jnp.dot(x2, w1t_ref[2 * F_IN:3 * F_IN, :], preferred_element_type=jnp.float32)
    a = jnp.maximum(h + b1_ref[...], 0.0).astype(jnp.bfloat16)
    d = jnp.dot(a, w2t_ref[...], preferred_element_type=jnp.float32) + b2_ref[...]
    d_ref[0] = d


def _k3_call(xr, w1t, b1r, w2t, b2r):
    BR = 8192
    NB = (M_MIX * R) // BR
    return pl.pallas_call(
        _k3_body,
        grid=(S, NB),
        in_specs=[
            pl.BlockSpec((1, 1, BR, F_IN), lambda s, t: (s, 0, t, 0)),
            pl.BlockSpec((1, 1, BR, F_IN), lambda s, t: (s, 1, t, 0)),
            pl.BlockSpec((1, 1, BR, F_IN), lambda s, t: (s, 2, t, 0)),
            pl.BlockSpec((3 * F_IN, HID), lambda s, t: (0, 0)),
            pl.BlockSpec((1, HID), lambda s, t: (0, 0)),
            pl.BlockSpec((HID, F_OUT), lambda s, t: (0, 0)),
            pl.BlockSpec((1, F_OUT), lambda s, t: (0, 0)),
        ],
        out_specs=pl.BlockSpec((1, BR, F_OUT), lambda s, t: (s, t, 0)),
        out_shape=jax.ShapeDtypeStruct((S, M_MIX * R, F_OUT), jnp.float32),
    )(xr, xr, xr, w1t, b1r, w2t, b2r)


def _k4_body(d_hbm, dst_hbm, out_hbm, acc_v, dch_v, ul_v, gb0, gb1, sm0, sm1):
    wid = lax.axis_index("s") * 2 + lax.axis_index("c")
    lane = lax.iota(jnp.int32, 16)
    DCH = 2048
    ACC = COLS_PER_TILE * ROWW

    def one_pass(p, _):
        c0 = p * (NW * COLS_PER_TILE) + wid * COLS_PER_TILE

        @plsc.parallel_loop(0, ACC // 16, 1, unroll=8)
        def _(k):
            acc_v[pl.ds(k * 16, 16)] = jnp.zeros((16,), jnp.float32)

        def comp_chunk(ch, cnt):
            pltpu.sync_copy(dst_hbm.at[pl.ds(ch * DCH, DCH)], dch_v)

            def comp_vec(j, cnt):
                d16 = dch_v[pl.ds(j * 16, 16)]
                rel = d16 - c0
                msk = (rel >= 0) & (rel < COLS_PER_TILE)
                pc = jnp.cumsum(jnp.where(msk, 1, 0))
                pos = pc + (cnt - 1)
                u = ch * DCH + j * 16 + lane
                plsc.store_scatter(ul_v, [pos], u * 32 + rel, mask=msk)
                return cnt + jnp.max(pc, axis=0)

            return lax.fori_loop(0, DCH // 16, comp_vec, cnt)

        cnt = lax.fori_loop(0, NU // DCH, comp_chunk, jnp.int32(0))
        nch = (cnt + 15) >> 4

        def load_rows(t):
            base = t * 16
            pk = ul_v[pl.ds(base, 16)]
            valid = (base + lane) < cnt
            pk = jnp.where(valid, pk, 0)
            u = pk >> 5
            row = (u // (3 * M_MIX)) * M_MIX + (u % M_MIX)
            return jnp.where(valid, row, 0)

        def g_start(t, buf, sem):
            pltpu.async_copy(d_hbm.at[load_rows(t)], buf, sem)

        def g_wait(buf, sem):
            pltpu.make_async_copy(d_hbm.at[pl.ds(0, 16)], buf, sem).wait()

        def rmw_chunk(t, buf):
            base = t * 16
            pk = ul_v[pl.ds(base, 16)]
            rel = pk & 31
            for g in range(16):
                @pl.when(base + g < cnt)
                def _():
                    c_g = jnp.max(jnp.where(lane == g, rel, 0), axis=0)
                    off = c_g * ROWW

                    @plsc.parallel_loop(0, ROWW // 16, 1, unroll=8)
                    def _(jj):
                        cur = acc_v[pl.ds(off + jj * 16, 16)]
                        dv = buf[g, jj // 8, pl.ds((jj % 8) * 16, 16)]
                        acc_v[pl.ds(off + jj * 16, 16)] = jnp.maximum(cur, dv)

        @pl.when(nch > 0)
        def _():
            g_start(0, gb0, sm0)

        npair = (nch + 1) >> 1

        def pair(q, _):
            t0 = 2 * q

            @pl.when(t0 + 1 < nch)
            def _():
                g_start(t0 + 1, gb1, sm1)

            g_wait(gb0, sm0)
            rmw_chunk(t0, gb0)

            @pl.when(t0 + 2 < nch)
            def _():
                g_start(t0 + 2, gb0, sm0)

            @pl.when(t0 + 1 < nch)
            def _():
                g_wait(gb1, sm1)
                rmw_chunk(t0 + 1, gb1)

            return 0

        lax.fori_loop(0, npair, pair, 0)
        pltpu.sync_copy(acc_v, out_hbm.at[pl.ds(c0 * ROWW, ACC)])
        return 0

    lax.fori_loop(0, NPASS, one_pass, 0)


def _k4_call(d, dst):
    mesh = plsc.VectorSubcoreMesh(core_axis_name="c", subcore_axis_name="s")
    f = functools.partial(
        pl.kernel,
        mesh=mesh,
        out_type=jax.ShapeDtypeStruct((C * ROWW,), jnp.float32),
        scratch_types=[
            pltpu.VMEM((COLS_PER_TILE * ROWW,), jnp.float32),
            pltpu.VMEM((2048,), jnp.int32),
            pltpu.VMEM((NU,), jnp.int32),
            pltpu.VMEM((16, 16, 128), jnp.float32),
            pltpu.VMEM((16, 16, 128), jnp.float32),
            pltpu.SemaphoreType.DMA,
            pltpu.SemaphoreType.DMA,
        ],
        compiler_params=pltpu.CompilerParams(needs_layout_passes=False),
    )(_k4_body)
    return f(d, dst)


def kernel(input_tensor, ind0, ind1, ind2, mix_ind, w1, b1, w2, b2):
    tin = jnp.transpose(input_tensor, (2, 1, 0)).reshape(C, 16, 128)
    tin = jnp.concatenate([tin, jnp.zeros((CP - C, 16, 128), jnp.float32)], axis=0)

    inds = jnp.stack([ind0, ind1, ind2], axis=1)
    a_l = inds[:, :, :, 0].reshape(NSET, 1, N_IND)
    b_l = inds[:, :, :, 1].reshape(NSET, 1, N_IND)
    b_c = inds[:, :, :, 1].reshape(NSET, N_IND, 1)
    q_c = mix_ind.reshape(NSET, M_MIX, 1)

    src6, dst6 = _k1_call(a_l, b_l, b_c, q_c)
    src = src6.reshape(NU)
    dst = dst6.reshape(NU)

    x = _k2_call(tin, src)
    xr = x.reshape(S, 3, M_MIX * R, F_IN)

    w1t = jnp.transpose(w1, (1, 0)).astype(jnp.bfloat16)
    w2t = jnp.transpose(w2, (1, 0)).astype(jnp.bfloat16)
    d = _k3_call(xr, w1t, b1.reshape(1, HID), w2t, b2.reshape(1, F_OUT))
    d2 = d.reshape(DROWS, 16, 128)

    out_flat = _k4_call(d2, dst)
    return jnp.transpose(out_flat.reshape(C, R, F_OUT), (2, 1, 0))
